# Initial kernel scaffold; baseline (speedup 1.0000x reference)
import jax
import jax.numpy as jnp
from jax.experimental import pallas as pl


def kernel(cat_features, interaction_table):
    raise NotImplementedError("write your pallas kernel here")


# speedup vs baseline: 2.0290x; 2.0290x over previous
import functools

import numpy as np
import jax
import jax.numpy as jnp
from jax import lax
from jax.experimental import pallas as pl
from jax.experimental.pallas import tpu as pltpu
from jax.experimental.pallas import tpu_sc as plsc

NCAT = 26
HASH = 100000
DM = 32
BATCH = 4096
NPAIR = NCAT * (NCAT - 1) // 2

_info = plsc.get_sparse_core_info()
_NC, _NS, _L = _info.num_cores, _info.num_subcores, _info.num_lanes
NW = _NC * _NS

ROWS = BATCH * NPAIR
RPW = ROWS // NW
BPW = BATCH // NW
STREAM = 128
SPC = 13
CHUNK = STREAM * SPC
NCHUNK = RPW // CHUNK

IPAD = 328

_i_np, _j_np = np.triu_indices(NCAT, k=1)
_II = np.zeros((IPAD,), np.int32)
_JJ = np.zeros((IPAD,), np.int32)
_II[:NPAIR] = _i_np
_JJ[:NPAIR] = _j_np


def _divmod_const(n, d):
    rinv = np.float32(1.0) / np.float32(d)
    q = (n.astype(jnp.float32) * rinv).astype(jnp.int32)
    r = n - q * d
    q = jnp.where(r >= d, q + 1, q)
    q = jnp.where(r < 0, q - 1, q)
    r = n - q * d
    return q, r


def _sc_body(cat_hbm, tab_hbm, ii_hbm, jj_hbm, out_hbm,
             cat_v, ii_v, jj_v, idx_v, rows_v, sem):
    wid = lax.axis_index("s") * _NC + lax.axis_index("c")
    pltpu.sync_copy(cat_hbm.at[pl.ds(wid * (BPW * NCAT), BPW * NCAT)], cat_v)
    pltpu.sync_copy(ii_hbm, ii_v)
    pltpu.sync_copy(jj_hbm, jj_v)
    lanes = lax.iota(jnp.int32, _L)

    def chunk_body(cidx, carry):
        n_base = cidx * CHUNK

        def comp(si, c):
            for v in range(STREAM // _L):
                n = n_base + si * STREAM + v * _L + lanes
                b_l, p = _divmod_const(n, NPAIR)
                ip = plsc.load_gather(ii_v, [p])
                jp = plsc.load_gather(jj_v, [p])
                base26 = b_l * NCAT
                ci = plsc.load_gather(cat_v, [base26 + ip])
                cj = plsc.load_gather(cat_v, [base26 + jp])
                _, h = _divmod_const(ci * 17 + cj * 31, HASH)
                idx_v[pl.ds(si * STREAM + v * _L, _L)] = h
            return c
        lax.fori_loop(0, SPC, comp, 0)

        def fire(si, c):
            iref = idx_v.at[pl.ds(si * STREAM, STREAM)]
            pltpu.async_copy(tab_hbm.at[iref],
                             rows_v.at[pl.ds(si * STREAM, STREAM)], sem)
            return c
        lax.fori_loop(0, SPC, fire, 0)

        row0 = wid * RPW + n_base
        pltpu.make_async_copy(out_hbm.at[pl.ds(row0, CHUNK)], rows_v, sem).wait()
        pltpu.sync_copy(rows_v, out_hbm.at[pl.ds(row0, CHUNK)])
        return carry

    lax.fori_loop(0, NCHUNK, chunk_body, 0)


_mesh = plsc.VectorSubcoreMesh(core_axis_name="c", subcore_axis_name="s")

_sc_kernel = functools.partial(
    pl.kernel,
    mesh=_mesh,
    out_type=jax.ShapeDtypeStruct((ROWS, DM), jnp.float32),
    scratch_types=[
        pltpu.VMEM((BPW * NCAT,), jnp.int32),
        pltpu.VMEM((IPAD,), jnp.int32),
        pltpu.VMEM((IPAD,), jnp.int32),
        pltpu.VMEM((CHUNK,), jnp.int32),
        pltpu.VMEM((CHUNK, DM), jnp.float32),
        pltpu.SemaphoreType.DMA,
    ],
    compiler_params=pltpu.CompilerParams(use_tc_tiling_on_sc=False,
                                         needs_layout_passes=False),
)(_sc_body)


def kernel(cat_features, interaction_table):
    cat_flat = cat_features.reshape(-1)
    ii = jnp.asarray(_II)
    jj = jnp.asarray(_JJ)
    out = _sc_kernel(cat_flat, interaction_table, ii, jj)
    return out.reshape(BATCH, NPAIR, DM)


# speedup vs baseline: 7.2275x; 3.5621x over previous
import functools

import numpy as np
import jax
import jax.numpy as jnp
from jax import lax
from jax.experimental import pallas as pl
from jax.experimental.pallas import tpu as pltpu
from jax.experimental.pallas import tpu_sc as plsc

NCAT = 26
HASH = 100000
DM = 32
BATCH = 4096
NPAIR = NCAT * (NCAT - 1) // 2

_info = plsc.get_sparse_core_info()
_NC, _NS, _L = _info.num_cores, _info.num_subcores, _info.num_lanes
NW = _NC * _NS

ROWS = BATCH * NPAIR
RPW = ROWS // NW
BPW = BATCH // NW
GB = 4
CROWS = GB * NPAIR
CPAD = 1312
NCHUNK = BPW // GB
NSTREAM = 10
TAIL = CPAD - NSTREAM * 128

IPAD = 328

_i_np, _j_np = np.triu_indices(NCAT, k=1)
_II = np.zeros((IPAD,), np.int32)
_JJ = np.zeros((IPAD,), np.int32)
_II[:NPAIR] = _i_np
_JJ[:NPAIR] = _j_np


def _divmod_const(n, d):
    rinv = np.float32(1.0) / np.float32(d)
    q = (n.astype(jnp.float32) * rinv).astype(jnp.int32)
    r = n - q * d
    q = jnp.where(r >= d, q + 1, q)
    q = jnp.where(r < 0, q - 1, q)
    r = n - q * d
    return q, r


def _sc_body(cat_hbm, tab_hbm, ii_hbm, jj_hbm, out_hbm,
             cat_v, ii_v, jj_v, idx_v, rows_v, gsem):
    wid = lax.axis_index("s") * _NC + lax.axis_index("c")
    pltpu.sync_copy(cat_hbm.at[pl.ds(wid * (BPW * NCAT), BPW * NCAT)], cat_v)
    pltpu.sync_copy(ii_hbm, ii_v)
    pltpu.sync_copy(jj_hbm, jj_v)
    lanes = lax.iota(jnp.int32, _L)

    def chunk_body(cidx, carry):
        n_base = cidx * CROWS

        def comp(k, c):
            for u in range(2):
                off = k * 2 * _L + u * _L
                n = jnp.minimum(n_base + off + lanes, RPW - 1)
                b_l, p = _divmod_const(n, NPAIR)
                ip = plsc.load_gather(ii_v, [p])
                jp = plsc.load_gather(jj_v, [p])
                base26 = b_l * NCAT
                ci = plsc.load_gather(cat_v, [base26 + ip])
                cj = plsc.load_gather(cat_v, [base26 + jp])
                _, h = _divmod_const(ci * 17 + cj * 31, HASH)
                idx_v[pl.ds(off, _L)] = h
            return c
        lax.fori_loop(0, CPAD // (2 * _L), comp, 0)

        def fire(si, c):
            iref = idx_v.at[pl.ds(si * 128, 128)]
            pltpu.async_copy(tab_hbm.at[iref],
                             rows_v.at[pl.ds(si * 128, 128)], gsem)
            return c
        lax.fori_loop(0, NSTREAM, fire, 0)
        tail_iref = idx_v.at[pl.ds(NSTREAM * 128, TAIL)]
        pltpu.async_copy(tab_hbm.at[tail_iref],
                         rows_v.at[pl.ds(NSTREAM * 128, TAIL)], gsem)

        def drain(si, c):
            pltpu.make_async_copy(
                tab_hbm.at[idx_v.at[pl.ds(si * 128, 128)]],
                rows_v.at[pl.ds(si * 128, 128)], gsem).wait()
            return c
        lax.fori_loop(0, NSTREAM, drain, 0)
        pltpu.make_async_copy(tab_hbm.at[tail_iref],
                              rows_v.at[pl.ds(NSTREAM * 128, TAIL)],
                              gsem).wait()

        b0 = wid * BPW + cidx * GB
        for b in range(GB):
            pltpu.sync_copy(rows_v.at[pl.ds(b * NPAIR, NPAIR)],
                            out_hbm.at[b0 + b])
        return carry

    lax.fori_loop(0, NCHUNK, chunk_body, 0)


_mesh = plsc.VectorSubcoreMesh(core_axis_name="c", subcore_axis_name="s")

_sc_kernel = functools.partial(
    pl.kernel,
    mesh=_mesh,
    out_type=jax.ShapeDtypeStruct((BATCH, NPAIR, DM), jnp.float32),
    scratch_types=[
        pltpu.VMEM((BPW * NCAT,), jnp.int32),
        pltpu.VMEM((IPAD,), jnp.int32),
        pltpu.VMEM((IPAD,), jnp.int32),
        pltpu.VMEM((CPAD,), jnp.int32),
        pltpu.VMEM((CPAD, DM), jnp.float32),
        pltpu.SemaphoreType.DMA,
    ],
    compiler_params=pltpu.CompilerParams(use_tc_tiling_on_sc=False,
                                         needs_layout_passes=False),
)(_sc_body)


def kernel(cat_features, interaction_table):
    cat_flat = cat_features.reshape(-1)
    ii = jnp.asarray(_II)
    jj = jnp.asarray(_JJ)
    return _sc_kernel(cat_flat, interaction_table, ii, jj)
